# baseline (speedup 1.0000x reference)
import functools

import jax
import jax.numpy as jnp
from jax import lax
from jax.experimental import pallas as pl
from jax.experimental.pallas import tpu as pltpu
from jax.experimental.pallas import tpu_sc as plsc

HT_H, HT_W = 240, 180
HW = HT_H * HT_W
SPHERE = 16384
NV = 262144
B, C = 4, 64

NC, NS, L = 2, 16, 16
CG = 32
G = (B * C) // CG
GPC = G // NC
K = 128
VPT = NV // NS
N_CHUNKS = VPT // K
ROWS_PER_TILE = SPHERE // NS

VMST = 1024
WOUT = 128

_mesh = plsc.VectorSubcoreMesh(core_axis_name="c", subcore_axis_name="s")


@functools.partial(
    pl.kernel,
    out_type=jax.ShapeDtypeStruct((B * C, SPHERE), jnp.float32),
    mesh=_mesh,
    scratch_types=[
        pltpu.VMEM_SHARED((SPHERE, CG), jnp.float32),
        pltpu.VMEM((K, CG), jnp.float32),
        pltpu.VMEM((K, CG), jnp.float32),
        pltpu.VMEM((VPT,), jnp.int32),
        pltpu.VMEM((N_CHUNKS, K), jnp.int32),
        pltpu.VMEM((VPT,), jnp.float32),
        pltpu.VMEM((VMST, 3), jnp.float32),
        pltpu.VMEM((WOUT, CG), jnp.float32),
        pltpu.VMEM((CG, WOUT), jnp.float32),
        pltpu.VMEM((128, CG), jnp.float32),
        pltpu.SemaphoreType.DMA,
        pltpu.SemaphoreType.DMA,
        pltpu.SemaphoreType.DMA,
        pltpu.SemaphoreType.DMA,
    ],
    compiler_params=pltpu.CompilerParams(
        needs_layout_passes=False, use_tc_tiling_on_sc=False
    ),
)
def _ht2sphere_sc(xT, vm, out, acc, rows0, rows1, htb, sphb, wb, vmst,
                  win, wout, zbuf, gsem0, gsem1, ssem0, ssem1):
    cid = lax.axis_index("c")
    sid = lax.axis_index("s")
    rows = (rows0, rows1)
    gsems = (gsem0, gsem1)
    ssems = (ssem0, ssem1)
    lanes = lax.iota(jnp.int32, L)

    vote_base = sid * VPT
    for st in range(VPT // VMST):
        pltpu.sync_copy(vm.at[pl.ds(vote_base + st * VMST, VMST)], vmst)

        def ebody(v, _):
            ridx = lanes + v * L
            ht_f = plsc.load_gather(vmst, [ridx, jnp.zeros((L,), jnp.int32)])
            w_f = plsc.load_gather(vmst, [ridx, jnp.ones((L,), jnp.int32)])
            s_f = plsc.load_gather(vmst, [ridx, jnp.full((L,), 2, jnp.int32)])
            base = st * VMST + v * L
            htb[pl.ds(base, L)] = ht_f.astype(jnp.int32)
            wb[pl.ds(base, L)] = w_f
            sphb[base // K, pl.ds(base % K, L)] = s_f.astype(jnp.int32)
            return 0

        lax.fori_loop(0, VMST // L, ebody, 0)

    def zrow(i, _):
        for j in range(CG // L):
            zbuf[i, pl.ds(j * L, L)] = jnp.zeros((L,), jnp.float32)
        return 0

    lax.fori_loop(0, 128, zrow, 0)

    for gl in range(GPC):
        group = cid * GPC + gl
        table = xT.at[group]

        def zstep(z, _):
            pltpu.sync_copy(
                zbuf, acc.at[pl.ds(sid * ROWS_PER_TILE + z * 128, 128)]
            )
            return 0

        lax.fori_loop(0, ROWS_PER_TILE // 128, zstep, 0)
        plsc.subcore_barrier()

        pltpu.async_copy(table.at[htb.at[pl.ds(0, K)]], rows0, gsem0)

        def pair_body(g, _):
            for par in range(2):
                ch = g * 2 + par
                buf, gsem, ssem = rows[par], gsems[par], ssems[par]
                nbuf, ngsem, nssem = rows[1 - par], gsems[1 - par], ssems[1 - par]

                @pl.when(ch >= 1)
                def _():
                    pltpu.make_async_copy(
                        nbuf, acc.at[sphb.at[ch - 1]], nssem
                    ).wait()

                @pl.when(ch + 1 < N_CHUNKS)
                def _():
                    pltpu.async_copy(
                        table.at[htb.at[pl.ds((ch + 1) * K, K)]], nbuf, ngsem
                    )

                pltpu.make_async_copy(
                    table.at[htb.at[pl.ds(ch * K, K)]], buf, gsem
                ).wait()

                @plsc.parallel_loop(0, K // L)
                def wblk(blkv):
                    v0 = ch * K + blkv * L
                    w16 = wb[pl.ds(v0, L)]
                    for l in range(L):
                        wv = jnp.take_along_axis(
                            w16, jnp.full((L,), l, jnp.int32), axis=0
                        )
                        r = blkv * L + l
                        for j in range(CG // L):
                            buf[r, pl.ds(j * L, L)] = (
                                buf[r, pl.ds(j * L, L)] * wv
                            )

                pltpu.async_copy(buf, acc.at[sphb.at[ch]], ssem, add=True)
            return 0

        lax.fori_loop(0, N_CHUNKS // 2, pair_body, 0)
        pltpu.make_async_copy(
            rows1, acc.at[sphb.at[N_CHUNKS - 1]], ssem1
        ).wait()
        plsc.subcore_barrier()

        for h in range(ROWS_PER_TILE // WOUT):
            pltpu.sync_copy(
                acc.at[pl.ds(sid * ROWS_PER_TILE + h * WOUT, WOUT)], win
            )

            def orow(r, _):
                for j in range(CG // L):
                    src = win[r, pl.ds(j * L, L)]
                    plsc.store_scatter(
                        wout,
                        [j * L + lanes, jnp.full((L,), r, jnp.int32)],
                        src,
                    )
                return 0

            lax.fori_loop(0, WOUT, orow, 0)
            pltpu.sync_copy(
                wout,
                out.at[
                    pl.ds(group * CG, CG),
                    pl.ds(sid * ROWS_PER_TILE + h * WOUT, WOUT),
                ],
            )
        plsc.subcore_barrier()


def kernel(input, vote_mapping):
    x = input.reshape(B, C // CG, CG, HW)
    xT = jnp.transpose(x, (0, 1, 3, 2)).reshape(G, HW, CG)
    out2d = _ht2sphere_sc(xT, vote_mapping)
    return out2d.reshape(B, C, SPHERE)


# speedup vs baseline: 1.3081x; 1.3081x over previous
import functools

import jax
import jax.numpy as jnp
from jax import lax
from jax.experimental import pallas as pl
from jax.experimental.pallas import tpu as pltpu
from jax.experimental.pallas import tpu_sc as plsc

HT_H, HT_W = 240, 180
HW = HT_H * HT_W
SPHERE = 16384
NV = 262144
B, C = 4, 64

NC, NS, L = 2, 16, 16
CG = 32
G = (B * C) // CG
GPC = G // NC
K = 128
VPT = NV // NS
N_CHUNKS = VPT // K
ROWS_PER_TILE = SPHERE // NS

_mesh = plsc.VectorSubcoreMesh(core_axis_name="c", subcore_axis_name="s")


@functools.partial(
    pl.kernel,
    out_type=jax.ShapeDtypeStruct((G, SPHERE, CG), jnp.float32),
    mesh=_mesh,
    scratch_types=[
        pltpu.VMEM_SHARED((SPHERE, CG), jnp.float32),
        pltpu.VMEM((K, CG), jnp.float32),
        pltpu.VMEM((K, CG), jnp.float32),
        pltpu.VMEM((VPT,), jnp.int32),
        pltpu.VMEM((N_CHUNKS, K), jnp.int32),
        pltpu.VMEM((VPT,), jnp.float32),
        pltpu.VMEM((ROWS_PER_TILE, CG), jnp.float32),
        pltpu.SemaphoreType.DMA,
        pltpu.SemaphoreType.DMA,
        pltpu.SemaphoreType.DMA,
        pltpu.SemaphoreType.DMA,
    ],
    compiler_params=pltpu.CompilerParams(
        needs_layout_passes=False, use_tc_tiling_on_sc=False
    ),
)
def _ht2sphere_sc(xT, ht, w, sph, out, acc, rows0, rows1, htb, sphb, wb,
                  zbuf, gsem0, gsem1, ssem0, ssem1):
    cid = lax.axis_index("c")
    sid = lax.axis_index("s")
    rows = (rows0, rows1)
    gsems = (gsem0, gsem1)
    ssems = (ssem0, ssem1)

    pltpu.sync_copy(ht.at[sid], htb)
    pltpu.sync_copy(w.at[sid], wb)
    pltpu.sync_copy(sph.at[sid], sphb)

    def zrow(i, _):
        for j in range(CG // L):
            zbuf[i, pl.ds(j * L, L)] = jnp.zeros((L,), jnp.float32)
        return 0

    lax.fori_loop(0, ROWS_PER_TILE, zrow, 0)

    for gl in range(GPC):
        group = cid * GPC + gl
        table = xT.at[group]

        pltpu.sync_copy(zbuf, acc.at[pl.ds(sid * ROWS_PER_TILE, ROWS_PER_TILE)])
        plsc.subcore_barrier()

        pltpu.async_copy(table.at[htb.at[pl.ds(0, K)]], rows0, gsem0)

        def pair_body(g, _):
            for par in range(2):
                ch = g * 2 + par
                buf, gsem, ssem = rows[par], gsems[par], ssems[par]
                nbuf, ngsem, nssem = rows[1 - par], gsems[1 - par], ssems[1 - par]

                @pl.when(ch >= 1)
                def _():
                    pltpu.make_async_copy(
                        nbuf, acc.at[sphb.at[ch - 1]], nssem
                    ).wait()

                @pl.when(ch + 1 < N_CHUNKS)
                def _():
                    pltpu.async_copy(
                        table.at[htb.at[pl.ds((ch + 1) * K, K)]], nbuf, ngsem
                    )

                pltpu.make_async_copy(
                    table.at[htb.at[pl.ds(ch * K, K)]], buf, gsem
                ).wait()

                @plsc.parallel_loop(0, K // L)
                def wblk(blk):
                    v0 = ch * K + blk * L
                    w16 = wb[pl.ds(v0, L)]
                    for l in range(L):
                        wv = jnp.take_along_axis(
                            w16, jnp.full((L,), l, jnp.int32), axis=0
                        )
                        r = blk * L + l
                        for j in range(CG // L):
                            buf[r, pl.ds(j * L, L)] = (
                                buf[r, pl.ds(j * L, L)] * wv
                            )

                pltpu.async_copy(buf, acc.at[sphb.at[ch]], ssem, add=True)
            return 0

        lax.fori_loop(0, N_CHUNKS // 2, pair_body, 0)

        pltpu.make_async_copy(
            rows1, acc.at[sphb.at[N_CHUNKS - 1]], ssem1
        ).wait()
        plsc.subcore_barrier()

        pltpu.sync_copy(
            acc.at[pl.ds(sid * ROWS_PER_TILE, ROWS_PER_TILE)],
            out.at[group].at[pl.ds(sid * ROWS_PER_TILE, ROWS_PER_TILE)],
        )
        plsc.subcore_barrier()


def kernel(input, vote_mapping):
    x = input.reshape(B, C // CG, CG, HW)
    xT = jnp.transpose(x, (0, 1, 3, 2)).reshape(G, HW, CG)
    ht = vote_mapping[:, 0].astype(jnp.int32).reshape(NS, VPT)
    w = vote_mapping[:, 1].reshape(NS, VPT)
    sph = vote_mapping[:, 2].astype(jnp.int32).reshape(NS, N_CHUNKS, K)
    outT = _ht2sphere_sc(xT, ht, w, sph)
    out = jnp.transpose(outT.reshape(B, C // CG, SPHERE, CG), (0, 1, 3, 2))
    return out.reshape(B, C, SPHERE)
